# Initial kernel scaffold; baseline (speedup 1.0000x reference)
import functools

import jax
import jax.numpy as jnp
from jax import lax
from jax.experimental import pallas as pl
from jax.experimental.pallas import tpu as pltpu
from jax.experimental.pallas import tpu_sc as plsc

NROWS = 100000
D = 128
NSEG = 1000
SEG_PAD = 1024
ROWS_PER_TILE = SEG_PAD // 16
C = 80
NCHUNKS = NROWS // C
NC = 2
NS = 16
CHUNKS_PER_CORE = NCHUNKS // NC


def _sc_segment_sums(X, gi, z_acc, z_cnt, ones):
    mesh = plsc.VectorSubcoreMesh(core_axis_name="c", subcore_axis_name="s")

    @functools.partial(
        pl.kernel,
        mesh=mesh,
        out_type=(
            jax.ShapeDtypeStruct((NC * SEG_PAD, D), jnp.float32),
            jax.ShapeDtypeStruct((NC * SEG_PAD, 16), jnp.float32),
        ),
        scratch_types=[
            pltpu.VMEM((C, D), jnp.float32),
            pltpu.VMEM((C,), jnp.int32),
            pltpu.VMEM((C, 16), jnp.float32),
            pltpu.VMEM((ROWS_PER_TILE, 16), jnp.float32),
            pltpu.VMEM_SHARED((SEG_PAD, D), jnp.float32),
            pltpu.VMEM_SHARED((SEG_PAD, 16), jnp.float32),
        ],
    )
    def k(x_hbm, gi_hbm, zacc_hbm, zcnt_hbm, ones_hbm, sums_hbm, cnts_hbm,
          x_v, idx_v, ones_v, st16_v, acc_s, cnt_s):
        cid = lax.axis_index("c")
        sid = lax.axis_index("s")
        seg_base = sid * ROWS_PER_TILE

        pltpu.sync_copy(zacc_hbm.at[pl.ds(seg_base, ROWS_PER_TILE)],
                        x_v.at[pl.ds(0, ROWS_PER_TILE)])
        pltpu.sync_copy(x_v.at[pl.ds(0, ROWS_PER_TILE)],
                        acc_s.at[pl.ds(seg_base, ROWS_PER_TILE)])
        pltpu.sync_copy(zcnt_hbm.at[pl.ds(seg_base, ROWS_PER_TILE)], st16_v)
        pltpu.sync_copy(st16_v, cnt_s.at[pl.ds(seg_base, ROWS_PER_TILE)])
        pltpu.sync_copy(ones_hbm, ones_v)
        plsc.subcore_barrier()

        start = cid * CHUNKS_PER_CORE + (sid * CHUNKS_PER_CORE) // NS
        end = cid * CHUNKS_PER_CORE + ((sid + 1) * CHUNKS_PER_CORE) // NS

        def body(kk, carry):
            base = kk * C
            pltpu.sync_copy(x_hbm.at[pl.ds(base, C)], x_v)
            pltpu.sync_copy(gi_hbm.at[pl.ds(base, C)], idx_v)
            pltpu.sync_copy(x_v, acc_s.at[idx_v], add=True)
            pltpu.sync_copy(ones_v, cnt_s.at[idx_v], add=True)
            return carry

        lax.fori_loop(start, end, body, 0)
        plsc.subcore_barrier()

        out_base = cid * SEG_PAD + seg_base
        pltpu.sync_copy(acc_s.at[pl.ds(seg_base, ROWS_PER_TILE)],
                        x_v.at[pl.ds(0, ROWS_PER_TILE)])
        pltpu.sync_copy(x_v.at[pl.ds(0, ROWS_PER_TILE)],
                        sums_hbm.at[pl.ds(out_base, ROWS_PER_TILE)])
        pltpu.sync_copy(cnt_s.at[pl.ds(seg_base, ROWS_PER_TILE)], st16_v)
        pltpu.sync_copy(st16_v, cnts_hbm.at[pl.ds(out_base, ROWS_PER_TILE)])

    return k(X, gi, z_acc, z_cnt, ones)


def _combine(sums, cnts):
    def body(s_ref, c_ref, o_ref):
        s = s_ref[0] + s_ref[1]
        c = c_ref[0] + c_ref[1]
        denom = jnp.maximum(c[:, 0:1], 1.0)
        o_ref[...] = s / denom

    return pl.pallas_call(
        body,
        out_shape=jax.ShapeDtypeStruct((SEG_PAD, D), jnp.float32),
    )(sums, cnts)


def kernel(X, graph_idx, n):
    gi = graph_idx.astype(jnp.int32)
    z_acc = jnp.zeros((SEG_PAD, D), jnp.float32)
    z_cnt = jnp.zeros((SEG_PAD, 16), jnp.float32)
    ones = jnp.ones((C, 16), jnp.float32)
    sums, cnts = _sc_segment_sums(X, gi, z_acc, z_cnt, ones)
    out = _combine(sums.reshape(NC, SEG_PAD, D), cnts.reshape(NC, SEG_PAD, 16))
    return out[:NSEG]


# speedup vs baseline: 4.0484x; 4.0484x over previous
import functools

import jax
import jax.numpy as jnp
from jax import lax
from jax.experimental import pallas as pl
from jax.experimental.pallas import tpu as pltpu
from jax.experimental.pallas import tpu_sc as plsc

NROWS = 100000
D = 128
NSEG = 1000
SEG_PAD = 1024
ROWS_PER_TILE = SEG_PAD // 16
C = 80
NCHUNKS = NROWS // C
NC = 2
NS = 16
CHUNKS_PER_CORE = NCHUNKS // NC


def _sc_segment_sums(X, gi, z_acc, ones):
    mesh = plsc.VectorSubcoreMesh(core_axis_name="c", subcore_axis_name="s")

    @functools.partial(
        pl.kernel,
        mesh=mesh,
        out_type=(
            jax.ShapeDtypeStruct((NC * SEG_PAD, D), jnp.float32),
            jax.ShapeDtypeStruct((NC * SEG_PAD, D), jnp.float32),
        ),
        scratch_types=[
            pltpu.VMEM((C, D), jnp.float32),
            pltpu.VMEM((C,), jnp.int32),
            pltpu.VMEM((C, D), jnp.float32),
            pltpu.VMEM_SHARED((SEG_PAD, D), jnp.float32),
            pltpu.VMEM_SHARED((SEG_PAD, D), jnp.float32),
        ],
    )
    def k(x_hbm, gi_hbm, zacc_hbm, ones_hbm, sums_hbm, cnts_hbm,
          x_v, idx_v, ones_v, acc_s, cnt_s):
        cid = lax.axis_index("c")
        sid = lax.axis_index("s")
        seg_base = sid * ROWS_PER_TILE

        pltpu.sync_copy(zacc_hbm.at[pl.ds(seg_base, ROWS_PER_TILE)],
                        x_v.at[pl.ds(0, ROWS_PER_TILE)])
        pltpu.sync_copy(x_v.at[pl.ds(0, ROWS_PER_TILE)],
                        acc_s.at[pl.ds(seg_base, ROWS_PER_TILE)])
        pltpu.sync_copy(x_v.at[pl.ds(0, ROWS_PER_TILE)],
                        cnt_s.at[pl.ds(seg_base, ROWS_PER_TILE)])
        pltpu.sync_copy(ones_hbm, ones_v)
        plsc.subcore_barrier()

        start = cid * CHUNKS_PER_CORE + (sid * CHUNKS_PER_CORE) // NS
        end = cid * CHUNKS_PER_CORE + ((sid + 1) * CHUNKS_PER_CORE) // NS

        def body(kk, carry):
            base = kk * C
            pltpu.sync_copy(x_hbm.at[pl.ds(base, C)], x_v)
            pltpu.sync_copy(gi_hbm.at[pl.ds(base, C)], idx_v)
            pltpu.sync_copy(x_v, acc_s.at[idx_v], add=True)
            pltpu.sync_copy(ones_v, cnt_s.at[idx_v], add=True)
            return carry

        lax.fori_loop(start, end, body, 0)
        plsc.subcore_barrier()

        out_base = cid * SEG_PAD + seg_base
        pltpu.sync_copy(acc_s.at[pl.ds(seg_base, ROWS_PER_TILE)],
                        x_v.at[pl.ds(0, ROWS_PER_TILE)])
        pltpu.sync_copy(x_v.at[pl.ds(0, ROWS_PER_TILE)],
                        sums_hbm.at[pl.ds(out_base, ROWS_PER_TILE)])
        pltpu.sync_copy(cnt_s.at[pl.ds(seg_base, ROWS_PER_TILE)],
                        ones_v.at[pl.ds(0, ROWS_PER_TILE)])
        pltpu.sync_copy(ones_v.at[pl.ds(0, ROWS_PER_TILE)],
                        cnts_hbm.at[pl.ds(out_base, ROWS_PER_TILE)])

    return k(X, gi, z_acc, ones)


def _combine(sums, cnts):
    def body(s_ref, c_ref, o_ref):
        s = s_ref[0] + s_ref[1]
        c = c_ref[0] + c_ref[1]
        denom = jnp.maximum(c[:, 0:1], 1.0)
        o_ref[...] = s / denom

    return pl.pallas_call(
        body,
        out_shape=jax.ShapeDtypeStruct((SEG_PAD, D), jnp.float32),
    )(sums, cnts)


def kernel(X, graph_idx, n):
    gi = graph_idx.astype(jnp.int32)
    z_acc = jnp.zeros((SEG_PAD, D), jnp.float32)
    ones = jnp.ones((C, D), jnp.float32)
    sums, cnts = _sc_segment_sums(X, gi, z_acc, ones)
    out = _combine(sums.reshape(NC, SEG_PAD, D), cnts.reshape(NC, SEG_PAD, D))
    return out[:NSEG]


# speedup vs baseline: 6.0685x; 1.4990x over previous
import functools

import jax
import jax.numpy as jnp
from jax import lax
from jax.experimental import pallas as pl
from jax.experimental.pallas import tpu as pltpu
from jax.experimental.pallas import tpu_sc as plsc

NROWS = 100000
D = 128
NSEG = 1000
SEG_PAD = 1024
ROWS_PER_TILE = SEG_PAD // 16
C = 128
NFULL = NROWS // C
TAIL = NROWS - NFULL * C
NC = 2
NS = 16


def _sc_segment_sums(X, gi, z_acc, ones):
    mesh = plsc.VectorSubcoreMesh(core_axis_name="c", subcore_axis_name="s")

    @functools.partial(
        pl.kernel,
        mesh=mesh,
        out_type=(
            jax.ShapeDtypeStruct((NC * SEG_PAD, D), jnp.float32),
            jax.ShapeDtypeStruct((NC * SEG_PAD, D), jnp.float32),
        ),
        scratch_types=[
            pltpu.VMEM((C, D), jnp.float32),
            pltpu.VMEM((C, D), jnp.float32),
            pltpu.VMEM((C,), jnp.int32),
            pltpu.VMEM((C,), jnp.int32),
            pltpu.VMEM((C, D), jnp.float32),
            pltpu.VMEM((TAIL, D), jnp.float32),
            pltpu.VMEM((TAIL,), jnp.int32),
            pltpu.VMEM_SHARED((SEG_PAD, D), jnp.float32),
            pltpu.VMEM_SHARED((SEG_PAD, D), jnp.float32),
            pltpu.SemaphoreType.DMA((2,)),
        ],
    )
    def k(x_hbm, gi_hbm, zacc_hbm, ones_hbm, sums_hbm, cnts_hbm,
          x_v0, x_v1, idx_v0, idx_v1, ones_v, tx_v, ti_v, acc_s, cnt_s,
          dma_sems):
        cid = lax.axis_index("c")
        sid = lax.axis_index("s")
        seg_base = sid * ROWS_PER_TILE
        x_bufs = (x_v0, x_v1)
        idx_bufs = (idx_v0, idx_v1)

        pltpu.sync_copy(zacc_hbm.at[pl.ds(seg_base, ROWS_PER_TILE)],
                        x_v0.at[pl.ds(0, ROWS_PER_TILE)])
        pltpu.sync_copy(x_v0.at[pl.ds(0, ROWS_PER_TILE)],
                        acc_s.at[pl.ds(seg_base, ROWS_PER_TILE)])
        pltpu.sync_copy(x_v0.at[pl.ds(0, ROWS_PER_TILE)],
                        cnt_s.at[pl.ds(seg_base, ROWS_PER_TILE)])
        pltpu.sync_copy(ones_hbm, ones_v)

        core_s = (cid * NFULL) // NC
        core_e = ((cid + 1) * NFULL) // NC
        clen = core_e - core_s
        start = core_s + (sid * clen) // NS
        end = core_s + ((sid + 1) * clen) // NS

        def start_dma(ck, b):
            base = ck * C
            pltpu.async_copy(x_hbm.at[pl.ds(base, C)], x_bufs[b],
                             dma_sems.at[b])
            pltpu.async_copy(gi_hbm.at[pl.ds(base, C)], idx_bufs[b],
                             dma_sems.at[b])

        def wait_dma(b):
            pltpu.make_async_copy(x_hbm.at[pl.ds(0, C)], x_bufs[b],
                                  dma_sems.at[b]).wait()
            pltpu.make_async_copy(gi_hbm.at[pl.ds(0, C)], idx_bufs[b],
                                  dma_sems.at[b]).wait()

        start_dma(start, 0)

        @pl.when(start + 1 < end)
        def _():
            start_dma(start + 1, 1)

        plsc.subcore_barrier()

        def body(i, carry):
            kk = start + 2 * i
            for b in (0, 1):
                cur = kk + b

                @pl.when(cur < end)
                def _():
                    wait_dma(b)
                    pltpu.sync_copy(x_bufs[b], acc_s.at[idx_bufs[b]],
                                    add=True)
                    pltpu.sync_copy(ones_v, cnt_s.at[idx_bufs[b]],
                                    add=True)

                    @pl.when(cur + 2 < end)
                    def _():
                        start_dma(cur + 2, b)
            return carry

        niter = (end - start + 1) // 2
        lax.fori_loop(0, niter, body, 0)

        @pl.when(jnp.logical_and(cid == NC - 1, sid == NS - 1))
        def _():
            pltpu.sync_copy(x_hbm.at[pl.ds(NFULL * C, TAIL)], tx_v)
            pltpu.sync_copy(gi_hbm.at[pl.ds(NFULL * C, TAIL)], ti_v)
            pltpu.sync_copy(tx_v, acc_s.at[ti_v], add=True)
            pltpu.sync_copy(ones_v.at[pl.ds(0, TAIL)], cnt_s.at[ti_v],
                            add=True)

        plsc.subcore_barrier()

        out_base = cid * SEG_PAD + seg_base
        pltpu.sync_copy(acc_s.at[pl.ds(seg_base, ROWS_PER_TILE)],
                        x_v0.at[pl.ds(0, ROWS_PER_TILE)])
        pltpu.sync_copy(x_v0.at[pl.ds(0, ROWS_PER_TILE)],
                        sums_hbm.at[pl.ds(out_base, ROWS_PER_TILE)])
        pltpu.sync_copy(cnt_s.at[pl.ds(seg_base, ROWS_PER_TILE)],
                        ones_v.at[pl.ds(0, ROWS_PER_TILE)])
        pltpu.sync_copy(ones_v.at[pl.ds(0, ROWS_PER_TILE)],
                        cnts_hbm.at[pl.ds(out_base, ROWS_PER_TILE)])

    return k(X, gi, z_acc, ones)


def _combine(sums, cnts):
    def body(s_ref, c_ref, o_ref):
        s = s_ref[0] + s_ref[1]
        c = c_ref[0] + c_ref[1]
        denom = jnp.maximum(c[:, 0:1], 1.0)
        o_ref[...] = s / denom

    return pl.pallas_call(
        body,
        out_shape=jax.ShapeDtypeStruct((SEG_PAD, D), jnp.float32),
    )(sums, cnts)


def kernel(X, graph_idx, n):
    gi = graph_idx.astype(jnp.int32)
    z_acc = jnp.zeros((SEG_PAD, D), jnp.float32)
    ones = jnp.ones((C, D), jnp.float32)
    sums, cnts = _sc_segment_sums(X, gi, z_acc, ones)
    out = _combine(sums.reshape(NC, SEG_PAD, D), cnts.reshape(NC, SEG_PAD, D))
    return out[:NSEG]


# speedup vs baseline: 6.2516x; 1.0302x over previous
import functools

import jax
import jax.numpy as jnp
from jax import lax
from jax.experimental import pallas as pl
from jax.experimental.pallas import tpu as pltpu
from jax.experimental.pallas import tpu_sc as plsc

NROWS = 100000
D = 128
NSEG = 1000
SEG_PAD = 1024
ROWS_PER_TILE = SEG_PAD // 16
C = 128
NFULL = NROWS // C
TAIL = NROWS - NFULL * C
NC = 2
NS = 16
NBUF = 4


def _sc_segment_sums(X, gi, z_acc, ones):
    mesh = plsc.VectorSubcoreMesh(core_axis_name="c", subcore_axis_name="s")

    @functools.partial(
        pl.kernel,
        mesh=mesh,
        out_type=(
            jax.ShapeDtypeStruct((NC * SEG_PAD, D), jnp.float32),
            jax.ShapeDtypeStruct((NC * SEG_PAD, D), jnp.float32),
        ),
        scratch_types=[
            pltpu.VMEM((NBUF, C, D), jnp.float32),
            pltpu.VMEM((NBUF, C), jnp.int32),
            pltpu.VMEM((C, D), jnp.float32),
            pltpu.VMEM((TAIL, D), jnp.float32),
            pltpu.VMEM((TAIL,), jnp.int32),
            pltpu.VMEM_SHARED((SEG_PAD, D), jnp.float32),
            pltpu.VMEM_SHARED((SEG_PAD, D), jnp.float32),
            pltpu.SemaphoreType.DMA((NBUF,)),
            pltpu.SemaphoreType.DMA((NBUF,)),
        ],
    )
    def k(x_hbm, gi_hbm, zacc_hbm, ones_hbm, sums_hbm, cnts_hbm,
          x_r, idx_r, ones_v, tx_v, ti_v, acc_s, cnt_s, in_sems, sc_sems):
        cid = lax.axis_index("c")
        sid = lax.axis_index("s")
        seg_base = sid * ROWS_PER_TILE

        stage = x_r.at[0, pl.ds(0, ROWS_PER_TILE)]
        pltpu.sync_copy(zacc_hbm.at[pl.ds(seg_base, ROWS_PER_TILE)], stage)
        pltpu.sync_copy(stage, acc_s.at[pl.ds(seg_base, ROWS_PER_TILE)])
        pltpu.sync_copy(stage, cnt_s.at[pl.ds(seg_base, ROWS_PER_TILE)])
        pltpu.sync_copy(ones_hbm, ones_v)

        core_s = (cid * NFULL) // NC
        core_e = ((cid + 1) * NFULL) // NC
        clen = core_e - core_s
        start = core_s + (sid * clen) // NS
        end = core_s + ((sid + 1) * clen) // NS
        n = end - start

        def start_dma(j, b):
            base = (start + j) * C
            pltpu.async_copy(x_hbm.at[pl.ds(base, C)], x_r.at[b],
                             in_sems.at[b])
            pltpu.async_copy(gi_hbm.at[pl.ds(base, C)], idx_r.at[b],
                             in_sems.at[b])

        def wait_dma(b):
            pltpu.make_async_copy(x_hbm.at[pl.ds(0, C)], x_r.at[b],
                                  in_sems.at[b]).wait()
            pltpu.make_async_copy(gi_hbm.at[pl.ds(0, C)], idx_r.at[b],
                                  in_sems.at[b]).wait()

        def wait_scat(b):
            pltpu.make_async_copy(x_r.at[b], acc_s.at[idx_r.at[b]],
                                  sc_sems.at[b]).wait()
            pltpu.make_async_copy(ones_v, cnt_s.at[idx_r.at[b]],
                                  sc_sems.at[b]).wait()

        start_dma(0, 0)
        start_dma(1, 1)
        plsc.subcore_barrier()

        def body(j, carry):
            b = j % NBUF
            b2 = (j + 2) % NBUF
            wait_dma(b)
            pltpu.async_copy(x_r.at[b], acc_s.at[idx_r.at[b]], sc_sems.at[b],
                             add=True)
            pltpu.async_copy(ones_v, cnt_s.at[idx_r.at[b]], sc_sems.at[b],
                             add=True)

            @pl.when(j + 2 < n)
            def _():
                @pl.when(j >= 2)
                def _():
                    wait_scat(b2)

                start_dma(j + 2, b2)

            return carry

        lax.fori_loop(0, n, body, 0)
        for t in range(NBUF):
            wait_scat((n - NBUF + t) % NBUF)

        @pl.when(jnp.logical_and(cid == NC - 1, sid == NS - 1))
        def _():
            pltpu.sync_copy(x_hbm.at[pl.ds(NFULL * C, TAIL)], tx_v)
            pltpu.sync_copy(gi_hbm.at[pl.ds(NFULL * C, TAIL)], ti_v)
            pltpu.sync_copy(tx_v, acc_s.at[ti_v], add=True)
            pltpu.sync_copy(ones_v.at[pl.ds(0, TAIL)], cnt_s.at[ti_v],
                            add=True)

        plsc.subcore_barrier()

        out_base = cid * SEG_PAD + seg_base
        pltpu.sync_copy(acc_s.at[pl.ds(seg_base, ROWS_PER_TILE)], stage)
        pltpu.sync_copy(stage, sums_hbm.at[pl.ds(out_base, ROWS_PER_TILE)])
        pltpu.sync_copy(cnt_s.at[pl.ds(seg_base, ROWS_PER_TILE)],
                        ones_v.at[pl.ds(0, ROWS_PER_TILE)])
        pltpu.sync_copy(ones_v.at[pl.ds(0, ROWS_PER_TILE)],
                        cnts_hbm.at[pl.ds(out_base, ROWS_PER_TILE)])

    return k(X, gi, z_acc, ones)


def _combine(sums, cnts):
    def body(s_ref, c_ref, o_ref):
        s = s_ref[0] + s_ref[1]
        c = c_ref[0] + c_ref[1]
        denom = jnp.maximum(c[:, 0:1], 1.0)
        o_ref[...] = s / denom

    return pl.pallas_call(
        body,
        out_shape=jax.ShapeDtypeStruct((SEG_PAD, D), jnp.float32),
    )(sums, cnts)


def kernel(X, graph_idx, n):
    gi = graph_idx.astype(jnp.int32)
    z_acc = jnp.zeros((SEG_PAD, D), jnp.float32)
    ones = jnp.ones((C, D), jnp.float32)
    sums, cnts = _sc_segment_sums(X, gi, z_acc, ones)
    out = _combine(sums.reshape(NC, SEG_PAD, D), cnts.reshape(NC, SEG_PAD, D))
    return out[:NSEG]


# speedup vs baseline: 6.5057x; 1.0407x over previous
import functools

import jax
import jax.numpy as jnp
from jax import lax
from jax.experimental import pallas as pl
from jax.experimental.pallas import tpu as pltpu
from jax.experimental.pallas import tpu_sc as plsc

NROWS = 100000
D = 128
NSEG = 1000
SEG_PAD = 1024
ROWS_PER_TILE = SEG_PAD // 16
CB = 256
NBIG = 99840 // CB
LEFT = 99840
TAIL = 99968
NC = 2
NS = 16
NBUF = 3
SHARD = 6256
SSTEPS = 13


def _sc_segment_sums(X, gi, gi2, z_acc):
    mesh = plsc.VectorSubcoreMesh(core_axis_name="c", subcore_axis_name="s")

    @functools.partial(
        pl.kernel,
        mesh=mesh,
        compiler_params=pltpu.CompilerParams(needs_layout_passes=False),
        out_type=(
            jax.ShapeDtypeStruct((NC * SEG_PAD, D), jnp.float32),
            jax.ShapeDtypeStruct((SEG_PAD,), jnp.float32),
        ),
        scratch_types=[
            pltpu.VMEM((NBUF, CB, D), jnp.float32),
            pltpu.VMEM((NBUF, 2, 128), jnp.int32),
            pltpu.VMEM((128,), jnp.int32),
            pltpu.VMEM((32, D), jnp.float32),
            pltpu.VMEM((32,), jnp.int32),
            pltpu.VMEM((SHARD,), jnp.int32),
            pltpu.VMEM((SEG_PAD + 16,), jnp.int32),
            pltpu.VMEM((SEG_PAD,), jnp.float32),
            pltpu.VMEM((16, ROWS_PER_TILE), jnp.float32),
            pltpu.VMEM_SHARED((SEG_PAD, D), jnp.float32),
            pltpu.VMEM_SHARED((16, SEG_PAD), jnp.float32),
            pltpu.SemaphoreType.DMA((NBUF,)),
            pltpu.SemaphoreType.DMA((NBUF,)),
        ],
    )
    def k(x_hbm, gi_hbm, gi2_hbm, zacc_hbm, sums_hbm, cnts_hbm,
          x_r, idx_r, li_v, tx_v, ti_v, shard_v, lb_v, cl_v, red_v,
          acc_s, cstage_s, in_sems, sc_sems):
        cid = lax.axis_index("c")
        sid = lax.axis_index("s")
        seg_base = sid * ROWS_PER_TILE

        stage = x_r.at[0, pl.ds(0, ROWS_PER_TILE)]
        pltpu.sync_copy(zacc_hbm.at[pl.ds(seg_base, ROWS_PER_TILE)], stage)
        pltpu.sync_copy(stage, acc_s.at[pl.ds(seg_base, ROWS_PER_TILE)])

        core_s = (cid * NBIG) // NC
        core_e = ((cid + 1) * NBIG) // NC
        clen = core_e - core_s
        start = core_s + (sid * clen) // NS
        end = core_s + ((sid + 1) * clen) // NS
        n = end - start

        def start_dma(j, b):
            base = (start + j) * CB
            pltpu.async_copy(x_hbm.at[pl.ds(base, CB)], x_r.at[b],
                             in_sems.at[b])
            pltpu.async_copy(gi2_hbm.at[pl.ds((start + j) * 2, 2)],
                             idx_r.at[b], in_sems.at[b])

        def wait_dma(b):
            pltpu.make_async_copy(x_hbm.at[pl.ds(0, CB)], x_r.at[b],
                                  in_sems.at[b]).wait()
            pltpu.make_async_copy(gi2_hbm.at[pl.ds(0, 2)], idx_r.at[b],
                                  in_sems.at[b]).wait()

        def scat_async(b):
            for g in (0, 1):
                pltpu.async_copy(x_r.at[b, pl.ds(g * 128, 128)],
                                 acc_s.at[idx_r.at[b, g]], sc_sems.at[b],
                                 add=True)

        def wait_scat(b):
            for g in (0, 1):
                pltpu.make_async_copy(x_r.at[b, pl.ds(g * 128, 128)],
                                      acc_s.at[idx_r.at[b, g]],
                                      sc_sems.at[b]).wait()

        start_dma(0, 0)
        start_dma(1, 1)
        plsc.subcore_barrier()

        def body(j, carry):
            b = j % NBUF
            b2 = (j + 2) % NBUF
            wait_dma(b)
            scat_async(b)

            @pl.when(j + 2 < n)
            def _():
                @pl.when(j >= 1)
                def _():
                    wait_scat(b2)

                start_dma(j + 2, b2)

            return carry

        lax.fori_loop(0, n, body, 0)
        for t in range(NBUF):
            wait_scat((n - NBUF + t) % NBUF)

        @pl.when(jnp.logical_and(cid == NC - 1, sid == NS - 1))
        def _():
            pltpu.sync_copy(x_hbm.at[pl.ds(LEFT, 128)],
                            x_r.at[0, pl.ds(0, 128)])
            pltpu.sync_copy(gi_hbm.at[pl.ds(LEFT, 128)], li_v)
            pltpu.sync_copy(x_r.at[0, pl.ds(0, 128)], acc_s.at[li_v],
                            add=True)
            pltpu.sync_copy(x_hbm.at[pl.ds(TAIL, 32)], tx_v)
            pltpu.sync_copy(gi_hbm.at[pl.ds(TAIL, 32)], ti_v)
            pltpu.sync_copy(tx_v, acc_s.at[ti_v], add=True)

        off = ((sid * 6250) // 8) * 8
        nxt = jnp.where(sid == NS - 1, NROWS,
                        (((sid + 1) * 6250) // 8) * 8)
        slen = nxt - off
        pltpu.sync_copy(gi_hbm.at[pl.ds(off, SHARD)], shard_v)
        lanes = lax.iota(jnp.int32, 16)

        def search_body(v, carry):
            svec = lanes + v * 16

            def step(_, lohi):
                lo, hi = lohi
                mid = (lo + hi) // 2
                g = plsc.load_gather(shard_v,
                                     [jnp.minimum(mid, SHARD - 1)])
                ge = g >= svec
                active = lo < hi
                lo = jnp.where(jnp.logical_and(active,
                                               jnp.logical_not(ge)),
                               mid + 1, lo)
                hi = jnp.where(jnp.logical_and(active, ge), mid, hi)
                return lo, hi

            lo0 = jnp.zeros((16,), jnp.int32)
            hi0 = jnp.zeros((16,), jnp.int32) + slen
            lo, hi = lax.fori_loop(0, SSTEPS, step, (lo0, hi0))
            lb_v[pl.ds(v * 16, 16)] = lo
            return carry

        lax.fori_loop(0, (SEG_PAD + 16) // 16, search_body, 0)

        def cnt_body(v, carry):
            hi_sh = plsc.load_gather(lb_v, [lanes + v * 16 + 1])
            lo_al = lb_v[pl.ds(v * 16, 16)]
            cl_v[pl.ds(v * 16, 16)] = (hi_sh - lo_al).astype(jnp.float32)
            return carry

        lax.fori_loop(0, SEG_PAD // 16, cnt_body, 0)
        pltpu.sync_copy(cl_v, cstage_s.at[sid])
        plsc.subcore_barrier()

        for r in range(16):
            pltpu.sync_copy(cstage_s.at[r, pl.ds(seg_base, ROWS_PER_TILE)],
                            red_v.at[r])
        for g in range(ROWS_PER_TILE // 16):
            tot = red_v[0, pl.ds(g * 16, 16)]
            for r in range(1, 16):
                tot = tot + red_v[r, pl.ds(g * 16, 16)]
            cl_v[pl.ds(g * 16, 16)] = tot

        @pl.when(cid == 0)
        def _():
            pltpu.sync_copy(cl_v.at[pl.ds(0, ROWS_PER_TILE)],
                            cnts_hbm.at[pl.ds(seg_base, ROWS_PER_TILE)])

        out_base = cid * SEG_PAD + seg_base
        pltpu.sync_copy(acc_s.at[pl.ds(seg_base, ROWS_PER_TILE)], stage)
        pltpu.sync_copy(stage, sums_hbm.at[pl.ds(out_base, ROWS_PER_TILE)])

    return k(X, gi, gi2, z_acc)


def _combine(sums, cnts):
    def body(s_ref, c_ref, o_ref):
        s = s_ref[0] + s_ref[1]
        denom = jnp.maximum(c_ref[...], 1.0)
        o_ref[...] = s / denom

    return pl.pallas_call(
        body,
        out_shape=jax.ShapeDtypeStruct((SEG_PAD, D), jnp.float32),
    )(sums, cnts)


def kernel(X, graph_idx, n):
    gi = graph_idx.astype(jnp.int32)
    gi2 = gi[:LEFT].reshape(LEFT // 128, 128)
    z_acc = jnp.zeros((SEG_PAD, D), jnp.float32)
    sums, cnts = _sc_segment_sums(X, gi, gi2, z_acc)
    out = _combine(sums.reshape(NC, SEG_PAD, D), cnts.reshape(SEG_PAD, 1))
    return out[:NSEG]


# speedup vs baseline: 7.8075x; 1.2001x over previous
import functools

import jax
import jax.numpy as jnp
from jax import lax
from jax.experimental import pallas as pl
from jax.experimental.pallas import tpu as pltpu
from jax.experimental.pallas import tpu_sc as plsc

NROWS = 100000
D = 128
NSEG = 1000
SEG_PAD = 1024
ROWS_PER_TILE = SEG_PAD // 16
C = 128
NFULL = NROWS // C
TAIL = NROWS - NFULL * C
NC = 2
NS = 16
NBUF = 4
HALF = NROWS // NC
SHARD = 3128
SSTEPS = 12


def _sc_segment_sums(X, gi, z_acc):
    mesh = plsc.VectorSubcoreMesh(core_axis_name="c", subcore_axis_name="s")

    @functools.partial(
        pl.kernel,
        mesh=mesh,
        compiler_params=pltpu.CompilerParams(needs_layout_passes=False),
        out_type=(
            jax.ShapeDtypeStruct((NC * SEG_PAD, D), jnp.float32),
            jax.ShapeDtypeStruct((NC * SEG_PAD,), jnp.float32),
        ),
        scratch_types=[
            pltpu.VMEM((NBUF, C, D), jnp.float32),
            pltpu.VMEM((NBUF, C), jnp.int32),
            pltpu.VMEM((TAIL, D), jnp.float32),
            pltpu.VMEM((TAIL,), jnp.int32),
            pltpu.VMEM((SHARD,), jnp.int32),
            pltpu.VMEM((SEG_PAD,), jnp.float32),
            pltpu.VMEM((16, ROWS_PER_TILE), jnp.float32),
            pltpu.VMEM_SHARED((SEG_PAD, D), jnp.float32),
            pltpu.VMEM_SHARED((16, SEG_PAD), jnp.float32),
            pltpu.SemaphoreType.DMA((NBUF,)),
            pltpu.SemaphoreType.DMA((NBUF,)),
        ],
    )
    def k(x_hbm, gi_hbm, zacc_hbm, sums_hbm, cnts_hbm,
          x_r, idx_r, tx_v, ti_v, shard_v, cl_v, red_v,
          acc_s, cstage_s, in_sems, sc_sems):
        cid = lax.axis_index("c")
        sid = lax.axis_index("s")
        seg_base = sid * ROWS_PER_TILE

        stage = x_r.at[0, pl.ds(0, ROWS_PER_TILE)]
        pltpu.sync_copy(zacc_hbm.at[pl.ds(seg_base, ROWS_PER_TILE)], stage)
        pltpu.sync_copy(stage, acc_s.at[pl.ds(seg_base, ROWS_PER_TILE)])

        core_s = (cid * NFULL) // NC
        core_e = ((cid + 1) * NFULL) // NC
        clen = core_e - core_s
        start = core_s + (sid * clen) // NS
        end = core_s + ((sid + 1) * clen) // NS
        n = end - start

        def start_dma(j, b):
            base = (start + j) * C
            pltpu.async_copy(x_hbm.at[pl.ds(base, C)], x_r.at[b],
                             in_sems.at[b])
            pltpu.async_copy(gi_hbm.at[pl.ds(base, C)], idx_r.at[b],
                             in_sems.at[b])

        def wait_dma(b):
            pltpu.make_async_copy(x_hbm.at[pl.ds(0, C)], x_r.at[b],
                                  in_sems.at[b]).wait()
            pltpu.make_async_copy(gi_hbm.at[pl.ds(0, C)], idx_r.at[b],
                                  in_sems.at[b]).wait()

        def wait_scat(b):
            pltpu.make_async_copy(x_r.at[b], acc_s.at[idx_r.at[b]],
                                  sc_sems.at[b]).wait()

        start_dma(0, 0)
        start_dma(1, 1)
        plsc.subcore_barrier()

        def body(j, carry):
            b = j % NBUF
            b2 = (j + 2) % NBUF
            wait_dma(b)
            pltpu.async_copy(x_r.at[b], acc_s.at[idx_r.at[b]], sc_sems.at[b],
                             add=True)

            @pl.when(j + 2 < n)
            def _():
                @pl.when(j >= 2)
                def _():
                    wait_scat(b2)

                start_dma(j + 2, b2)

            return carry

        lax.fori_loop(0, n, body, 0)
        for t in range(NBUF):
            wait_scat((n - NBUF + t) % NBUF)

        @pl.when(jnp.logical_and(cid == NC - 1, sid == NS - 1))
        def _():
            pltpu.sync_copy(x_hbm.at[pl.ds(NFULL * C, TAIL)], tx_v)
            pltpu.sync_copy(gi_hbm.at[pl.ds(NFULL * C, TAIL)], ti_v)
            pltpu.sync_copy(tx_v, acc_s.at[ti_v], add=True)

        off = cid * HALF + ((sid * 3125) // 8) * 8
        nxt = cid * HALF + jnp.where(sid == NS - 1, HALF,
                                     (((sid + 1) * 3125) // 8) * 8)
        slen = nxt - off
        pltpu.sync_copy(gi_hbm.at[pl.ds(off, SHARD)], shard_v)
        lanes = lax.iota(jnp.int32, 16)

        def zero_body(v, carry):
            cl_v[pl.ds(v * 16, 16)] = jnp.zeros((16,), jnp.float32)
            return carry

        lax.fori_loop(0, SEG_PAD // 16, zero_body, 0)

        def search16(svec):
            def step(_, lohi):
                lo, hi = lohi
                mid = (lo + hi) // 2
                g = plsc.load_gather(shard_v,
                                     [jnp.minimum(mid, SHARD - 1)])
                ge = g >= svec
                active = lo < hi
                lo = jnp.where(jnp.logical_and(active,
                                               jnp.logical_not(ge)),
                               mid + 1, lo)
                hi = jnp.where(jnp.logical_and(active, ge), mid, hi)
                return lo, hi

            lo0 = jnp.zeros((16,), jnp.int32)
            hi0 = jnp.zeros((16,), jnp.int32) + slen
            lo, _ = lax.fori_loop(0, SSTEPS, step, (lo0, hi0))
            return lo

        first = jnp.min(shard_v[pl.ds(0, 16)])
        last_ub = jnp.max(shard_v[pl.ds(SHARD - 16, 16)])

        def search_body(v, carry):
            svec = lanes + v * 16
            lb_lo = search16(svec)
            lb_hi = search16(svec + 1)
            cl_v[pl.ds(v * 16, 16)] = (lb_hi - lb_lo).astype(jnp.float32)
            return carry

        lax.fori_loop(first // 16, last_ub // 16 + 1, search_body, 0)
        pltpu.sync_copy(cl_v, cstage_s.at[sid])
        plsc.subcore_barrier()

        for r in range(16):
            pltpu.sync_copy(cstage_s.at[r, pl.ds(seg_base, ROWS_PER_TILE)],
                            red_v.at[r])
        for g in range(ROWS_PER_TILE // 16):
            tot = red_v[0, pl.ds(g * 16, 16)]
            for r in range(1, 16):
                tot = tot + red_v[r, pl.ds(g * 16, 16)]
            cl_v[pl.ds(g * 16, 16)] = tot

        out_base = cid * SEG_PAD + seg_base
        pltpu.sync_copy(cl_v.at[pl.ds(0, ROWS_PER_TILE)],
                        cnts_hbm.at[pl.ds(out_base, ROWS_PER_TILE)])

        pltpu.sync_copy(acc_s.at[pl.ds(seg_base, ROWS_PER_TILE)], stage)
        pltpu.sync_copy(stage, sums_hbm.at[pl.ds(out_base, ROWS_PER_TILE)])

    return k(X, gi, z_acc)


def _combine(sums, cnts):
    def body(s_ref, c_ref, o_ref):
        s = s_ref[0] + s_ref[1]
        c = c_ref[0] + c_ref[1]
        denom = jnp.maximum(c, 1.0)
        o_ref[...] = s / denom

    return pl.pallas_call(
        body,
        out_shape=jax.ShapeDtypeStruct((SEG_PAD, D), jnp.float32),
    )(sums, cnts)


def kernel(X, graph_idx, n):
    gi = graph_idx.astype(jnp.int32)
    z_acc = jnp.zeros((SEG_PAD, D), jnp.float32)
    sums, cnts = _sc_segment_sums(X, gi, z_acc)
    out = _combine(sums.reshape(NC, SEG_PAD, D), cnts.reshape(NC, SEG_PAD, 1))
    return out[:NSEG]


# speedup vs baseline: 8.0675x; 1.0333x over previous
import functools

import jax
import jax.numpy as jnp
from jax import lax
from jax.experimental import pallas as pl
from jax.experimental.pallas import tpu as pltpu
from jax.experimental.pallas import tpu_sc as plsc

NROWS = 100000
D = 128
NSEG = 1000
SEG_PAD = 1024
ROWS_PER_TILE = SEG_PAD // 16
C = 128
NFULL = NROWS // C
TAIL = NROWS - NFULL * C
NC = 2
NS = 16
NBUF = 5
HALF = NROWS // NC
SHARD = 3128
SSTEPS = 12


def _sc_segment_sums(X, gi, z_acc):
    mesh = plsc.VectorSubcoreMesh(core_axis_name="c", subcore_axis_name="s")

    @functools.partial(
        pl.kernel,
        mesh=mesh,
        compiler_params=pltpu.CompilerParams(needs_layout_passes=False),
        out_type=(
            jax.ShapeDtypeStruct((NC * SEG_PAD, D), jnp.float32),
            jax.ShapeDtypeStruct((NC * SEG_PAD,), jnp.float32),
        ),
        scratch_types=[
            pltpu.VMEM((NBUF, C, D), jnp.float32),
            pltpu.VMEM((NBUF, C), jnp.int32),
            pltpu.VMEM((TAIL, D), jnp.float32),
            pltpu.VMEM((TAIL,), jnp.int32),
            pltpu.VMEM((SHARD,), jnp.int32),
            pltpu.VMEM((SEG_PAD,), jnp.float32),
            pltpu.VMEM((16, ROWS_PER_TILE), jnp.float32),
            pltpu.VMEM_SHARED((SEG_PAD, D), jnp.float32),
            pltpu.VMEM_SHARED((16, SEG_PAD), jnp.float32),
            pltpu.SemaphoreType.DMA((NBUF,)),
            pltpu.SemaphoreType.DMA((NBUF,)),
        ],
    )
    def k(x_hbm, gi_hbm, zacc_hbm, sums_hbm, cnts_hbm,
          x_r, idx_r, tx_v, ti_v, shard_v, cl_v, red_v,
          acc_s, cstage_s, in_sems, sc_sems):
        cid = lax.axis_index("c")
        sid = lax.axis_index("s")
        seg_base = sid * ROWS_PER_TILE

        stage = x_r.at[0, pl.ds(0, ROWS_PER_TILE)]
        pltpu.sync_copy(zacc_hbm.at[pl.ds(seg_base, ROWS_PER_TILE)], stage)
        pltpu.sync_copy(stage, acc_s.at[pl.ds(seg_base, ROWS_PER_TILE)])

        core_s = (cid * NFULL) // NC
        core_e = ((cid + 1) * NFULL) // NC
        clen = core_e - core_s
        start = core_s + (sid * clen) // NS
        end = core_s + ((sid + 1) * clen) // NS
        n = end - start

        def start_dma(j, b):
            base = (start + j) * C
            pltpu.async_copy(x_hbm.at[pl.ds(base, C)], x_r.at[b],
                             in_sems.at[b])
            pltpu.async_copy(gi_hbm.at[pl.ds(base, C)], idx_r.at[b],
                             in_sems.at[b])

        def wait_dma(b):
            pltpu.make_async_copy(x_hbm.at[pl.ds(0, C)], x_r.at[b],
                                  in_sems.at[b]).wait()
            pltpu.make_async_copy(gi_hbm.at[pl.ds(0, C)], idx_r.at[b],
                                  in_sems.at[b]).wait()

        def wait_scat(b):
            pltpu.make_async_copy(x_r.at[b], acc_s.at[idx_r.at[b]],
                                  sc_sems.at[b]).wait()

        start_dma(0, 0)
        start_dma(1, 1)
        start_dma(2, 2)
        plsc.subcore_barrier()

        def body(j, carry):
            b = j % NBUF
            b2 = (j + 3) % NBUF
            wait_dma(b)
            pltpu.async_copy(x_r.at[b], acc_s.at[idx_r.at[b]], sc_sems.at[b],
                             add=True)

            @pl.when(j + 3 < n)
            def _():
                @pl.when(j >= 2)
                def _():
                    wait_scat(b2)

                start_dma(j + 3, b2)

            return carry

        lax.fori_loop(0, n, body, 0)
        for t in range(NBUF):
            wait_scat((n - NBUF + t) % NBUF)

        @pl.when(jnp.logical_and(cid == NC - 1, sid == NS - 1))
        def _():
            pltpu.sync_copy(x_hbm.at[pl.ds(NFULL * C, TAIL)], tx_v)
            pltpu.sync_copy(gi_hbm.at[pl.ds(NFULL * C, TAIL)], ti_v)
            pltpu.sync_copy(tx_v, acc_s.at[ti_v], add=True)

        off = cid * HALF + ((sid * 3125) // 8) * 8
        nxt = cid * HALF + jnp.where(sid == NS - 1, HALF,
                                     (((sid + 1) * 3125) // 8) * 8)
        slen = nxt - off
        pltpu.sync_copy(gi_hbm.at[pl.ds(off, SHARD)], shard_v)
        lanes = lax.iota(jnp.int32, 16)

        def zero_body(v, carry):
            cl_v[pl.ds(v * 16, 16)] = jnp.zeros((16,), jnp.float32)
            return carry

        lax.fori_loop(0, SEG_PAD // 16, zero_body, 0)

        def search16(svec):
            def step(_, lohi):
                lo, hi = lohi
                mid = (lo + hi) // 2
                g = plsc.load_gather(shard_v,
                                     [jnp.minimum(mid, SHARD - 1)])
                ge = g >= svec
                active = lo < hi
                lo = jnp.where(jnp.logical_and(active,
                                               jnp.logical_not(ge)),
                               mid + 1, lo)
                hi = jnp.where(jnp.logical_and(active, ge), mid, hi)
                return lo, hi

            lo0 = jnp.zeros((16,), jnp.int32)
            hi0 = jnp.zeros((16,), jnp.int32) + slen
            lo, _ = lax.fori_loop(0, SSTEPS, step, (lo0, hi0))
            return lo

        first = jnp.min(shard_v[pl.ds(0, 16)])
        last_ub = jnp.max(shard_v[pl.ds(SHARD - 16, 16)])

        def search_body(v, carry):
            svec = lanes + v * 16
            lb_lo = search16(svec)
            lb_hi = search16(svec + 1)
            cl_v[pl.ds(v * 16, 16)] = (lb_hi - lb_lo).astype(jnp.float32)
            return carry

        lax.fori_loop(first // 16, last_ub // 16 + 1, search_body, 0)
        pltpu.sync_copy(cl_v, cstage_s.at[sid])
        plsc.subcore_barrier()

        for r in range(16):
            pltpu.sync_copy(cstage_s.at[r, pl.ds(seg_base, ROWS_PER_TILE)],
                            red_v.at[r])
        for g in range(ROWS_PER_TILE // 16):
            tot = red_v[0, pl.ds(g * 16, 16)]
            for r in range(1, 16):
                tot = tot + red_v[r, pl.ds(g * 16, 16)]
            cl_v[pl.ds(g * 16, 16)] = tot

        out_base = cid * SEG_PAD + seg_base
        pltpu.sync_copy(cl_v.at[pl.ds(0, ROWS_PER_TILE)],
                        cnts_hbm.at[pl.ds(out_base, ROWS_PER_TILE)])

        pltpu.sync_copy(acc_s.at[pl.ds(seg_base, ROWS_PER_TILE)], stage)
        pltpu.sync_copy(stage, sums_hbm.at[pl.ds(out_base, ROWS_PER_TILE)])

    return k(X, gi, z_acc)


def _combine(sums, cnts):
    def body(s_ref, c_ref, o_ref):
        s = s_ref[0] + s_ref[1]
        c = c_ref[0] + c_ref[1]
        denom = jnp.maximum(c, 1.0)
        o_ref[...] = s / denom

    return pl.pallas_call(
        body,
        out_shape=jax.ShapeDtypeStruct((SEG_PAD, D), jnp.float32),
    )(sums, cnts)


def kernel(X, graph_idx, n):
    gi = graph_idx.astype(jnp.int32)
    z_acc = jnp.zeros((SEG_PAD, D), jnp.float32)
    sums, cnts = _sc_segment_sums(X, gi, z_acc)
    out = _combine(sums.reshape(NC, SEG_PAD, D), cnts.reshape(NC, SEG_PAD, 1))
    return out[:NSEG]


# speedup vs baseline: 8.1505x; 1.0103x over previous
import functools

import jax
import jax.numpy as jnp
from jax import lax
from jax.experimental import pallas as pl
from jax.experimental.pallas import tpu as pltpu
from jax.experimental.pallas import tpu_sc as plsc

NROWS = 100000
D = 128
NSEG = 1000
SEG_PAD = 1024
ROWS_PER_TILE = SEG_PAD // 16
C = 128
NFULL = NROWS // C
TAIL = NROWS - NFULL * C
NC = 2
NS = 16
NBUF = 6
HALF = NROWS // NC
SHARD = 3128
SSTEPS = 12


def _sc_segment_sums(X, gi, z_acc):
    mesh = plsc.VectorSubcoreMesh(core_axis_name="c", subcore_axis_name="s")

    @functools.partial(
        pl.kernel,
        mesh=mesh,
        compiler_params=pltpu.CompilerParams(needs_layout_passes=False),
        out_type=(
            jax.ShapeDtypeStruct((NC * SEG_PAD, D), jnp.float32),
            jax.ShapeDtypeStruct((NC * SEG_PAD,), jnp.float32),
        ),
        scratch_types=[
            pltpu.VMEM((NBUF, C, D), jnp.float32),
            pltpu.VMEM((NBUF, C), jnp.int32),
            pltpu.VMEM((TAIL, D), jnp.float32),
            pltpu.VMEM((TAIL,), jnp.int32),
            pltpu.VMEM((SHARD,), jnp.int32),
            pltpu.VMEM((SEG_PAD,), jnp.float32),
            pltpu.VMEM((16, ROWS_PER_TILE), jnp.float32),
            pltpu.VMEM_SHARED((SEG_PAD, D), jnp.float32),
            pltpu.VMEM_SHARED((16, SEG_PAD), jnp.float32),
            pltpu.SemaphoreType.DMA((NBUF,)),
            pltpu.SemaphoreType.DMA((NBUF,)),
        ],
    )
    def k(x_hbm, gi_hbm, zacc_hbm, sums_hbm, cnts_hbm,
          x_r, idx_r, tx_v, ti_v, shard_v, cl_v, red_v,
          acc_s, cstage_s, in_sems, sc_sems):
        cid = lax.axis_index("c")
        sid = lax.axis_index("s")
        seg_base = sid * ROWS_PER_TILE

        stage = x_r.at[0, pl.ds(0, ROWS_PER_TILE)]
        pltpu.sync_copy(zacc_hbm.at[pl.ds(seg_base, ROWS_PER_TILE)], stage)
        pltpu.sync_copy(stage, acc_s.at[pl.ds(seg_base, ROWS_PER_TILE)])

        core_s = (cid * NFULL) // NC
        core_e = ((cid + 1) * NFULL) // NC
        clen = core_e - core_s
        start = core_s + (sid * clen) // NS
        end = core_s + ((sid + 1) * clen) // NS
        n = end - start

        def start_dma(j, b):
            base = (start + j) * C
            pltpu.async_copy(x_hbm.at[pl.ds(base, C)], x_r.at[b],
                             in_sems.at[b])
            pltpu.async_copy(gi_hbm.at[pl.ds(base, C)], idx_r.at[b],
                             in_sems.at[b])

        def wait_dma(b):
            pltpu.make_async_copy(x_hbm.at[pl.ds(0, C)], x_r.at[b],
                                  in_sems.at[b]).wait()
            pltpu.make_async_copy(gi_hbm.at[pl.ds(0, C)], idx_r.at[b],
                                  in_sems.at[b]).wait()

        def wait_scat(b):
            pltpu.make_async_copy(x_r.at[b], acc_s.at[idx_r.at[b]],
                                  sc_sems.at[b]).wait()

        start_dma(0, 0)
        start_dma(1, 1)
        start_dma(2, 2)
        start_dma(3, 3)
        plsc.subcore_barrier()

        def body(j, carry):
            b = j % NBUF
            b2 = (j + 4) % NBUF
            wait_dma(b)
            pltpu.async_copy(x_r.at[b], acc_s.at[idx_r.at[b]], sc_sems.at[b],
                             add=True)

            @pl.when(j + 4 < n)
            def _():
                @pl.when(j >= 2)
                def _():
                    wait_scat(b2)

                start_dma(j + 4, b2)

            return carry

        lax.fori_loop(0, n, body, 0)
        for t in range(NBUF):
            wait_scat((n - NBUF + t) % NBUF)

        @pl.when(jnp.logical_and(cid == NC - 1, sid == NS - 1))
        def _():
            pltpu.sync_copy(x_hbm.at[pl.ds(NFULL * C, TAIL)], tx_v)
            pltpu.sync_copy(gi_hbm.at[pl.ds(NFULL * C, TAIL)], ti_v)
            pltpu.sync_copy(tx_v, acc_s.at[ti_v], add=True)

        off = cid * HALF + ((sid * 3125) // 8) * 8
        nxt = cid * HALF + jnp.where(sid == NS - 1, HALF,
                                     (((sid + 1) * 3125) // 8) * 8)
        slen = nxt - off
        pltpu.sync_copy(gi_hbm.at[pl.ds(off, SHARD)], shard_v)
        lanes = lax.iota(jnp.int32, 16)

        def zero_body(v, carry):
            cl_v[pl.ds(v * 16, 16)] = jnp.zeros((16,), jnp.float32)
            return carry

        lax.fori_loop(0, SEG_PAD // 16, zero_body, 0)

        def search16(svec):
            def step(_, lohi):
                lo, hi = lohi
                mid = (lo + hi) // 2
                g = plsc.load_gather(shard_v,
                                     [jnp.minimum(mid, SHARD - 1)])
                ge = g >= svec
                active = lo < hi
                lo = jnp.where(jnp.logical_and(active,
                                               jnp.logical_not(ge)),
                               mid + 1, lo)
                hi = jnp.where(jnp.logical_and(active, ge), mid, hi)
                return lo, hi

            lo0 = jnp.zeros((16,), jnp.int32)
            hi0 = jnp.zeros((16,), jnp.int32) + slen
            lo, _ = lax.fori_loop(0, SSTEPS, step, (lo0, hi0))
            return lo

        first = jnp.min(shard_v[pl.ds(0, 16)])
        last_ub = jnp.max(shard_v[pl.ds(SHARD - 16, 16)])

        def search_body(v, carry):
            svec = lanes + v * 16
            lb_lo = search16(svec)
            lb_hi = search16(svec + 1)
            cl_v[pl.ds(v * 16, 16)] = (lb_hi - lb_lo).astype(jnp.float32)
            return carry

        lax.fori_loop(first // 16, last_ub // 16 + 1, search_body, 0)
        pltpu.sync_copy(cl_v, cstage_s.at[sid])
        plsc.subcore_barrier()

        for r in range(16):
            pltpu.sync_copy(cstage_s.at[r, pl.ds(seg_base, ROWS_PER_TILE)],
                            red_v.at[r])
        for g in range(ROWS_PER_TILE // 16):
            tot = red_v[0, pl.ds(g * 16, 16)]
            for r in range(1, 16):
                tot = tot + red_v[r, pl.ds(g * 16, 16)]
            cl_v[pl.ds(g * 16, 16)] = tot

        out_base = cid * SEG_PAD + seg_base
        pltpu.sync_copy(cl_v.at[pl.ds(0, ROWS_PER_TILE)],
                        cnts_hbm.at[pl.ds(out_base, ROWS_PER_TILE)])

        pltpu.sync_copy(acc_s.at[pl.ds(seg_base, ROWS_PER_TILE)], stage)
        pltpu.sync_copy(stage, sums_hbm.at[pl.ds(out_base, ROWS_PER_TILE)])

    return k(X, gi, z_acc)


def _combine(sums, cnts):
    def body(s_ref, c_ref, o_ref):
        s = s_ref[0] + s_ref[1]
        c = c_ref[0] + c_ref[1]
        denom = jnp.maximum(c, 1.0)
        o_ref[...] = s / denom

    return pl.pallas_call(
        body,
        out_shape=jax.ShapeDtypeStruct((SEG_PAD, D), jnp.float32),
    )(sums, cnts)


def kernel(X, graph_idx, n):
    gi = graph_idx.astype(jnp.int32)
    z_acc = jnp.zeros((SEG_PAD, D), jnp.float32)
    sums, cnts = _sc_segment_sums(X, gi, z_acc)
    out = _combine(sums.reshape(NC, SEG_PAD, D), cnts.reshape(NC, SEG_PAD, 1))
    return out[:NSEG]
